# Initial kernel scaffold; baseline (speedup 1.0000x reference)
import jax
import jax.numpy as jnp
from jax.experimental import pallas as pl


def kernel(x, edge_index, batch, W1, b1, Wmu, bmu, Wls, bls):
    raise NotImplementedError("write your pallas kernel here")


# speedup vs baseline: 3.2727x; 3.2727x over previous
import functools

import jax
import jax.numpy as jnp
from jax import lax
from jax.experimental import pallas as pl
from jax.experimental.pallas import tpu as pltpu
from jax.experimental.pallas import tpu_sc as plsc

N = 10000
E = 320000
B = 256
D_IN = 128
D_HID = 128
MAX_LOGSTD = 10.0

N_P = 10240
RB = 512


def _k2_body(x_ref, dg_ref, o_ref):
    deg = 1.0 + dg_ref[0, :, 0] + dg_ref[1, :, 0]
    dis = lax.rsqrt(deg)[:, None]
    xs = x_ref[...] * dis
    o_ref[0] = xs[:, :64]
    o_ref[1] = xs[:, 64:]


def _k2(x_p, degs):
    return pl.pallas_call(
        _k2_body,
        grid=(N_P // RB,),
        in_specs=[
            pl.BlockSpec((RB, 128), lambda i: (i, 0)),
            pl.BlockSpec((2, RB, 16), lambda i: (0, i, 0)),
        ],
        out_specs=pl.BlockSpec((2, RB, 64), lambda i: (0, i, 0)),
        out_shape=jax.ShapeDtypeStruct((2, N_P, 64), jnp.float32),
    )(x_p, degs)


def _k4_body(y_ref, dg_ref, w_ref, b_ref, o_ref):
    deg = 1.0 + dg_ref[0, :, 0] + dg_ref[1, :, 0]
    dis = lax.rsqrt(deg)[:, None]
    y1 = jnp.concatenate([y_ref[0], y_ref[1]], axis=1) * dis
    h = jnp.maximum(jnp.dot(y1, w_ref[...],
                            preferred_element_type=jnp.float32) + b_ref[...], 0.0)
    hs = h * dis
    o_ref[0] = hs[:, :128]
    o_ref[1] = hs[:, 128:]


def _k4(y1_st, degs, W1, b1):
    return pl.pallas_call(
        _k4_body,
        grid=(N_P // RB,),
        in_specs=[
            pl.BlockSpec((2, RB, 64), lambda i: (0, i, 0)),
            pl.BlockSpec((2, RB, 16), lambda i: (0, i, 0)),
            pl.BlockSpec((128, 256), lambda i: (0, 0)),
            pl.BlockSpec((1, 256), lambda i: (0, 0)),
        ],
        out_specs=pl.BlockSpec((2, RB, 128), lambda i: (0, i, 0)),
        out_shape=jax.ShapeDtypeStruct((2, N_P, 128), jnp.float32),
    )(y1_st, degs, W1, b1.reshape(1, 256))


def _k6_body(y_ref, dg_ref, wm_ref, bm_ref, wl_ref, bl_ref, o_ref):
    deg = 1.0 + dg_ref[0, :, 0] + dg_ref[1, :, 0]
    dis = lax.rsqrt(deg)[:, None]
    ah = jnp.concatenate([y_ref[0], y_ref[1]], axis=1) * dis
    mu = jnp.dot(ah, wm_ref[...], preferred_element_type=jnp.float32) + bm_ref[...]
    ls = jnp.minimum(
        jnp.dot(ah, wl_ref[...], preferred_element_type=jnp.float32) + bl_ref[...],
        MAX_LOGSTD)
    o_ref[...] = jnp.concatenate([mu, ls], axis=1)


def _k6(y2_st, degs, Wmu, bmu, Wls, bls):
    return pl.pallas_call(
        _k6_body,
        grid=(N_P // RB,),
        in_specs=[
            pl.BlockSpec((2, RB, 128), lambda i: (0, i, 0)),
            pl.BlockSpec((2, RB, 16), lambda i: (0, i, 0)),
            pl.BlockSpec((256, 128), lambda i: (0, 0)),
            pl.BlockSpec((1, 128), lambda i: (0, 0)),
            pl.BlockSpec((256, 128), lambda i: (0, 0)),
            pl.BlockSpec((1, 128), lambda i: (0, 0)),
        ],
        out_specs=pl.BlockSpec((RB, 256), lambda i: (i, 0)),
        out_shape=jax.ShapeDtypeStruct((N_P, 256), jnp.float32),
    )(y2_st, degs, Wmu, bmu.reshape(1, 128), Wls, bls.reshape(1, 128))


def _deg_jnp(dst):
    cnt = jax.ops.segment_sum(jnp.ones((E,), jnp.float32), dst, num_segments=N_P)
    t = jnp.broadcast_to(cnt[:, None], (N_P, 16))
    z = jnp.zeros((N_P, 16), jnp.float32)
    return jnp.stack([t, z])


def _agg_jnp(vs_st, src, dst):
    v = jnp.concatenate([vs_st[0], vs_st[1]], axis=1)
    y = v + jax.ops.segment_sum(v[src], dst, num_segments=N_P)
    w2 = vs_st.shape[2]
    return jnp.stack([y[:, :w2], y[:, w2:]])


def _segmax_jnp(ml, batch_p):
    return jax.ops.segment_max(ml, batch_p, num_segments=B + 1)[:B]


def kernel(x, edge_index, batch, W1, b1, Wmu, bmu, Wls, bls):
    src = edge_index[0]
    dst = edge_index[1]
    x_p = jnp.pad(x, ((0, N_P - N), (0, 0)))
    batch_p = jnp.pad(batch, (0, N_P - N), constant_values=B)

    degs = _deg_jnp(dst)
    xs_st = _k2(x_p, degs)
    y1_st = _agg_jnp(xs_st, src, dst)
    hs_st = _k4(y1_st, degs, W1, b1)
    y2_st = _agg_jnp(hs_st, src, dst)
    ml = _k6(y2_st, degs, Wmu, bmu, Wls, bls)
    ml_g = _segmax_jnp(ml, batch_p)
    return (ml_g[:, :128], ml_g[:, 128:])


# speedup vs baseline: 6.4985x; 1.9857x over previous
import functools

import jax
import jax.numpy as jnp
from jax import lax
from jax.experimental import pallas as pl
from jax.experimental.pallas import tpu as pltpu
from jax.experimental.pallas import tpu_sc as plsc

N = 10000
E = 320000
B = 256
D_IN = 128
D_HID = 128
MAX_LOGSTD = 10.0

N_P = 10240
RB = 512


def _k2_body(x_ref, dg_ref, o_ref):
    deg = 1.0 + dg_ref[0, :, 0]
    dis = lax.rsqrt(deg)[:, None]
    o_ref[...] = x_ref[...] * dis


def _k2(x_p, degs):
    return pl.pallas_call(
        _k2_body,
        grid=(N_P // RB,),
        in_specs=[
            pl.BlockSpec((RB, 128), lambda i: (i, 0)),
            pl.BlockSpec((1, RB, 16), lambda i: (i // 10, i % 10, 0)),
        ],
        out_specs=pl.BlockSpec((RB, 128), lambda i: (i, 0)),
        out_shape=jax.ShapeDtypeStruct((N_P, 128), jnp.float32),
    )(x_p, degs)


def _k4_body(y_ref, dg_ref, w_ref, b_ref, o_ref):
    deg = 1.0 + dg_ref[0, :, 0]
    dis = lax.rsqrt(deg)[:, None]
    y1 = y_ref[0] * dis
    h = jnp.maximum(jnp.dot(y1, w_ref[...],
                            preferred_element_type=jnp.float32) + b_ref[...], 0.0)
    hs = h * dis
    o_ref[0] = hs[:, :128]
    o_ref[1] = hs[:, 128:]


def _k4(y1_st, degs, W1, b1):
    return pl.pallas_call(
        _k4_body,
        grid=(N_P // RB,),
        in_specs=[
            pl.BlockSpec((1, RB, 128), lambda i: (i // 10, i % 10, 0)),
            pl.BlockSpec((1, RB, 16), lambda i: (i // 10, i % 10, 0)),
            pl.BlockSpec((128, 256), lambda i: (0, 0)),
            pl.BlockSpec((1, 256), lambda i: (0, 0)),
        ],
        out_specs=pl.BlockSpec((2, RB, 128), lambda i: (0, i, 0)),
        out_shape=jax.ShapeDtypeStruct((2, N_P, 128), jnp.float32),
    )(y1_st, degs, W1, b1.reshape(1, 256))


def _k6_body(y_ref, dg_ref, wm_ref, bm_ref, wl_ref, bl_ref, o_ref):
    deg = 1.0 + dg_ref[0, :, 0]
    dis = lax.rsqrt(deg)[:, None]
    ah = jnp.concatenate([y_ref[0], y_ref[1]], axis=1) * dis
    mu = jnp.dot(ah, wm_ref[...], preferred_element_type=jnp.float32) + bm_ref[...]
    ls = jnp.minimum(
        jnp.dot(ah, wl_ref[...], preferred_element_type=jnp.float32) + bl_ref[...],
        MAX_LOGSTD)
    o_ref[...] = jnp.concatenate([mu, ls], axis=1)


def _k6(y2_st, degs, Wmu, bmu, Wls, bls):
    return pl.pallas_call(
        _k6_body,
        grid=(N_P // RB,),
        in_specs=[
            pl.BlockSpec((2, RB, 128), lambda i: (0, i, 0)),
            pl.BlockSpec((1, RB, 16), lambda i: (i // 10, i % 10, 0)),
            pl.BlockSpec((256, 128), lambda i: (0, 0)),
            pl.BlockSpec((1, 128), lambda i: (0, 0)),
            pl.BlockSpec((256, 128), lambda i: (0, 0)),
            pl.BlockSpec((1, 128), lambda i: (0, 0)),
        ],
        out_specs=pl.BlockSpec((RB, 256), lambda i: (i, 0)),
        out_shape=jax.ShapeDtypeStruct((N_P, 256), jnp.float32),
    )(y2_st, degs, Wmu, bmu.reshape(1, 128), Wls, bls.reshape(1, 128))


NC = 2
NS = 16
ECH = 128
K16 = 160
IBLK = 16
NBLK = K16 // IBLK
E16 = NS * K16 * ECH
HALF = N_P // 2
AR = HALF + 128


def _agg_sc(table, init, sidx2, didx2, nch, ir, init_in_table=False):
    rpt = ir // NS
    mesh = plsc.VectorSubcoreMesh(core_axis_name="c", subcore_axis_name="s")

    def _body(vs_hbm, in_hbm, si_hbm, di_hbm, out_hbm, si_v, di_v, buf, y_sh, sem):
        cid = lax.axis_index("c")
        sid = lax.axis_index("s")
        if in_hbm is None:
            pltpu.sync_copy(vs_hbm.at[pl.ds(cid * ir + sid * rpt, rpt)],
                            y_sh.at[pl.ds(sid * rpt, rpt)])
        else:
            pltpu.sync_copy(in_hbm.at[cid, pl.ds(sid * rpt, rpt)],
                            y_sh.at[pl.ds(sid * rpt, rpt)])
        plsc.subcore_barrier()

        def blk(bj, carry):
            pltpu.sync_copy(si_hbm.at[cid, sid, pl.ds(bj * IBLK, IBLK)], si_v)
            pltpu.sync_copy(di_hbm.at[cid, sid, pl.ds(bj * IBLK, IBLK)], di_v)

            def body(j, c2):
                pltpu.async_copy(vs_hbm.at[si_v.at[j]], buf, sem).wait()
                pltpu.sync_copy(buf, y_sh.at[di_v.at[j]], add=True)
                return c2

            lax.fori_loop(0, IBLK, body, 0)
            return carry

        lax.fori_loop(0, nch // IBLK, blk, 0)
        plsc.subcore_barrier()
        pltpu.sync_copy(y_sh.at[pl.ds(sid * rpt, rpt)],
                        out_hbm.at[cid, pl.ds(sid * rpt, rpt)])

    scratch = [
        pltpu.VMEM((IBLK, ECH), jnp.int32),
        pltpu.VMEM((IBLK, ECH), jnp.int32),
        pltpu.VMEM((ECH, 128), jnp.float32),
        pltpu.VMEM_SHARED((ir, 128), jnp.float32),
        pltpu.SemaphoreType.DMA,
    ]
    out_t = jax.ShapeDtypeStruct((2, ir, 128), jnp.float32)
    if init_in_table:
        def k_body(vs, si, di, out, *s):
            _body(vs, None, si, di, out, *s)
        k = pl.kernel(k_body, out_type=out_t, mesh=mesh, scratch_types=scratch)
        return k(table, sidx2, didx2)
    k = pl.kernel(_body, out_type=out_t, mesh=mesh, scratch_types=scratch)
    return k(table, init, sidx2, didx2)


def _deg_sc(didx2, zinit):
    rpt = AR // NS
    mesh = plsc.VectorSubcoreMesh(core_axis_name="c", subcore_axis_name="s")

    @functools.partial(
        pl.kernel,
        out_type=jax.ShapeDtypeStruct((2, AR, 16), jnp.float32),
        mesh=mesh,
        scratch_types=[
            pltpu.VMEM((K16, ECH), jnp.int32),
            pltpu.VMEM((ECH, 16), jnp.float32),
            pltpu.VMEM_SHARED((AR, 16), jnp.float32),
        ],
    )
    def k(di_hbm, z_hbm, out_hbm, di_v, ones_v, dg_sh):
        cid = lax.axis_index("c")
        sid = lax.axis_index("s")
        pltpu.sync_copy(di_hbm.at[cid, sid], di_v)

        def fill(i, carry):
            ones_v[i] = jnp.full((16,), 1.0, jnp.float32)
            return carry

        lax.fori_loop(0, ECH, fill, 0)
        pltpu.sync_copy(z_hbm.at[cid, pl.ds(sid * rpt, rpt)],
                        dg_sh.at[pl.ds(sid * rpt, rpt)])
        plsc.subcore_barrier()

        def body(j, carry):
            pltpu.sync_copy(ones_v, dg_sh.at[di_v.at[j]], add=True)
            return carry

        lax.fori_loop(0, K16, body, 0)
        plsc.subcore_barrier()
        pltpu.sync_copy(dg_sh.at[pl.ds(sid * rpt, rpt)],
                        out_hbm.at[cid, pl.ds(sid * rpt, rpt)])

    return k(didx2, zinit)


def _k7_body(ml_ref, ids_ref, o_ref):
    i = pl.program_id(0)

    @pl.when(i == 0)
    def _():
        o_ref[...] = jnp.full((B, 256), -jnp.inf, jnp.float32)

    ninf = jnp.float32(-jnp.inf)
    iota24 = lax.broadcasted_iota(jnp.int32, (1, 24), 1)
    for g in range(RB // 8):
        row0 = 8 * g
        rr, cc = row0 // 128, row0 % 128
        id0 = ids_ref[0, rr, cc]
        w0 = pl.multiple_of(jnp.minimum((id0 // 8) * 8, B - 24), 8)
        rel = ids_ref[0, rr, pl.ds(cc, 8)] - w0
        m = rel[:, None] == iota24
        rows_g = ml_ref[pl.ds(row0, 8), :]
        cs = [jnp.max(jnp.where(m[:, s:s + 1], rows_g, ninf), axis=0,
                      keepdims=True) for s in range(24)]
        cwin = jnp.concatenate(cs, axis=0)
        o_ref[pl.ds(w0, 24), :] = jnp.maximum(o_ref[pl.ds(w0, 24), :], cwin)


def _k7(ml, bat2):
    return pl.pallas_call(
        _k7_body,
        grid=(N_P // RB,),
        in_specs=[
            pl.BlockSpec((RB, 256), lambda i: (i, 0)),
            pl.BlockSpec((1, RB // 128, 128), lambda i: (i, 0, 0)),
        ],
        out_specs=pl.BlockSpec((B, 256), lambda i: (0, 0)),
        out_shape=jax.ShapeDtypeStruct((B, 256), jnp.float32),
    )(ml, bat2)


def kernel(x, edge_index, batch, W1, b1, Wmu, bmu, Wls, bls):
    src = edge_index[0]
    dst = edge_index[1]
    x_p = jnp.pad(x, ((0, N_P - N), (0, 0)))
    batch_p = jnp.pad(batch, (0, N_P - N), constant_values=B)

    padE = jnp.full((E16 - E,), N, jnp.int32)
    srcB = jnp.concatenate([src, padE]).reshape(NS, K16, ECH)
    dst_p = jnp.concatenate([dst, padE])
    dstB = dst_p.reshape(NS, K16, ECH)
    sidxA = jnp.stack([srcB, srcB])
    dloc0 = jnp.where(dst_p < HALF, dst_p, HALF)
    dloc1 = jnp.where(dst_p >= HALF, dst_p - HALF, HALF)
    dloc1 = jnp.where(dst_p < N_P, dloc1, HALF)
    didxA = jnp.stack([dloc0.reshape(NS, K16, ECH), dloc1.reshape(NS, K16, ECH)])
    sidxB = jnp.stack([srcB, srcB + N_P])
    didxB = jnp.stack([dstB, dstB])

    degs = _deg_sc(didxA, jnp.zeros((2, AR, 16), jnp.float32))
    xs = _k2(x_p, degs)
    initA = jnp.pad(xs.reshape(2, HALF, 128), ((0, 0), (0, 128), (0, 0)))
    y1_st = _agg_sc(xs, initA, sidxA, didxA, K16, AR)
    hs_st = _k4(y1_st, degs, W1, b1)
    y2_st = _agg_sc(hs_st.reshape(2 * N_P, 128), None, sidxB, didxB,
                    K16, N_P, init_in_table=True)
    ml = _k6(y2_st, degs, Wmu, bmu, Wls, bls)
    ml_g = _k7(ml, batch_p.reshape(N_P // RB, RB // 128, 128))
    return (ml_g[:, :128], ml_g[:, 128:])


# speedup vs baseline: 7.3595x; 1.1325x over previous
import functools

import jax
import jax.numpy as jnp
from jax import lax
from jax.experimental import pallas as pl
from jax.experimental.pallas import tpu as pltpu
from jax.experimental.pallas import tpu_sc as plsc

N = 10000
E = 320000
B = 256
D_IN = 128
D_HID = 128
MAX_LOGSTD = 10.0

N_P = 10240
RB = 512


def _k2_body(x_ref, dg_ref, o_ref):
    deg = 1.0 + dg_ref[0, :, 0]
    dis = lax.rsqrt(deg)[:, None]
    o_ref[...] = x_ref[...] * dis


def _k2(x_p, degs):
    return pl.pallas_call(
        _k2_body,
        grid=(N_P // RB,),
        in_specs=[
            pl.BlockSpec((RB, 128), lambda i: (i, 0)),
            pl.BlockSpec((1, RB, 16), lambda i: (i // 10, i % 10, 0)),
        ],
        out_specs=pl.BlockSpec((RB, 128), lambda i: (i, 0)),
        out_shape=jax.ShapeDtypeStruct((N_P, 128), jnp.float32),
    )(x_p, degs)


def _k4_body(y_ref, dg_ref, w_ref, b_ref, o_ref):
    deg = 1.0 + dg_ref[0, :, 0]
    dis = lax.rsqrt(deg)[:, None]
    y1 = (y_ref[0] + y_ref[1]) * dis
    h = jnp.maximum(jnp.dot(y1, w_ref[...],
                            preferred_element_type=jnp.float32) + b_ref[...], 0.0)
    hs = h * dis
    o_ref[0] = hs[:, :128]
    o_ref[1] = hs[:, 128:]


def _k4(y1_st, degs, W1, b1):
    return pl.pallas_call(
        _k4_body,
        grid=(N_P // RB,),
        in_specs=[
            pl.BlockSpec((2, RB, 128), lambda i: (0, i, 0)),
            pl.BlockSpec((1, RB, 16), lambda i: (i // 10, i % 10, 0)),
            pl.BlockSpec((128, 256), lambda i: (0, 0)),
            pl.BlockSpec((1, 256), lambda i: (0, 0)),
        ],
        out_specs=pl.BlockSpec((2, RB, 128), lambda i: (0, i, 0)),
        out_shape=jax.ShapeDtypeStruct((2, N_P, 128), jnp.float32),
    )(y1_st, degs, W1, b1.reshape(1, 256))


def _k6_body(y_ref, dg_ref, wm_ref, bm_ref, wl_ref, bl_ref, o_ref):
    deg = 1.0 + dg_ref[0, :, 0]
    dis = lax.rsqrt(deg)[:, None]
    ah = jnp.concatenate([y_ref[0], y_ref[1]], axis=1) * dis
    mu = jnp.dot(ah, wm_ref[...], preferred_element_type=jnp.float32) + bm_ref[...]
    ls = jnp.minimum(
        jnp.dot(ah, wl_ref[...], preferred_element_type=jnp.float32) + bl_ref[...],
        MAX_LOGSTD)
    o_ref[...] = jnp.concatenate([mu, ls], axis=1)


def _k6(y2_st, degs, Wmu, bmu, Wls, bls):
    return pl.pallas_call(
        _k6_body,
        grid=(N_P // RB,),
        in_specs=[
            pl.BlockSpec((2, RB, 128), lambda i: (0, i, 0)),
            pl.BlockSpec((1, RB, 16), lambda i: (i // 10, i % 10, 0)),
            pl.BlockSpec((256, 128), lambda i: (0, 0)),
            pl.BlockSpec((1, 128), lambda i: (0, 0)),
            pl.BlockSpec((256, 128), lambda i: (0, 0)),
            pl.BlockSpec((1, 128), lambda i: (0, 0)),
        ],
        out_specs=pl.BlockSpec((RB, 256), lambda i: (i, 0)),
        out_shape=jax.ShapeDtypeStruct((N_P, 256), jnp.float32),
    )(y2_st, degs, Wmu, bmu.reshape(1, 128), Wls, bls.reshape(1, 128))


NC = 2
NS = 16
ECH = 128
K16 = 160
IBLK = 16
NBLK = K16 // IBLK
E16 = NS * K16 * ECH
HALF = N_P // 2
AR = HALF + 128


def _agg_sc(table, init, sidx2, didx2, nch, ir, init_in_table=False):
    rpt = ir // NS
    mesh = plsc.VectorSubcoreMesh(core_axis_name="c", subcore_axis_name="s")

    def _body(vs_hbm, in_hbm, si_hbm, di_hbm, out_hbm, si_v, di_v, buf, buf2,
              y_sh, sem):
        cid = lax.axis_index("c")
        sid = lax.axis_index("s")
        if in_hbm is None:
            pltpu.sync_copy(vs_hbm.at[pl.ds(cid * ir + sid * rpt, rpt)],
                            y_sh.at[pl.ds(sid * rpt, rpt)])
        else:
            pltpu.sync_copy(in_hbm.at[cid, pl.ds(sid * rpt, rpt)],
                            y_sh.at[pl.ds(sid * rpt, rpt)])
        plsc.subcore_barrier()

        def blk(bj, carry):
            pltpu.sync_copy(si_hbm.at[cid, sid, pl.ds(bj * IBLK, IBLK)], si_v)
            pltpu.sync_copy(di_hbm.at[cid, sid, pl.ds(bj * IBLK, IBLK)], di_v)
            def body(j, c2):
                pltpu.async_copy(vs_hbm.at[si_v.at[j]], buf, sem).wait()
                pltpu.sync_copy(buf, y_sh.at[di_v.at[j]], add=True)
                return c2

            lax.fori_loop(0, IBLK, body, 0)
            return carry

        lax.fori_loop(0, nch // IBLK, blk, 0)
        plsc.subcore_barrier()
        pltpu.sync_copy(y_sh.at[pl.ds(sid * rpt, rpt)],
                        out_hbm.at[cid, pl.ds(sid * rpt, rpt)])

    scratch = [
        pltpu.VMEM((IBLK, ECH), jnp.int32),
        pltpu.VMEM((IBLK, ECH), jnp.int32),
        pltpu.VMEM((ECH, 128), jnp.float32),
        pltpu.VMEM((ECH, 128), jnp.float32),
        pltpu.VMEM_SHARED((ir, 128), jnp.float32),
        pltpu.SemaphoreType.DMA,
    ]
    out_t = jax.ShapeDtypeStruct((2, ir, 128), jnp.float32)
    if init_in_table:
        def k_body(vs, si, di, out, *s):
            _body(vs, None, si, di, out, *s)
        k = pl.kernel(k_body, out_type=out_t, mesh=mesh, scratch_types=scratch)
        return k(table, sidx2, didx2)
    k = pl.kernel(_body, out_type=out_t, mesh=mesh, scratch_types=scratch)
    return k(table, init, sidx2, didx2)


def _deg_sc(didx2, zinit):
    rpt = AR // NS
    mesh = plsc.VectorSubcoreMesh(core_axis_name="c", subcore_axis_name="s")

    @functools.partial(
        pl.kernel,
        out_type=jax.ShapeDtypeStruct((2, AR, 16), jnp.float32),
        mesh=mesh,
        scratch_types=[
            pltpu.VMEM((K16, ECH), jnp.int32),
            pltpu.VMEM((ECH, 16), jnp.float32),
            pltpu.VMEM_SHARED((AR, 16), jnp.float32),
        ],
    )
    def k(di_hbm, z_hbm, out_hbm, di_v, ones_v, dg_sh):
        cid = lax.axis_index("c")
        sid = lax.axis_index("s")
        pltpu.sync_copy(di_hbm.at[cid, sid], di_v)

        def fill(i, carry):
            ones_v[i] = jnp.full((16,), 1.0, jnp.float32)
            return carry

        lax.fori_loop(0, ECH, fill, 0)
        pltpu.sync_copy(z_hbm.at[cid, pl.ds(sid * rpt, rpt)],
                        dg_sh.at[pl.ds(sid * rpt, rpt)])
        plsc.subcore_barrier()

        def body(j, carry):
            pltpu.sync_copy(ones_v, dg_sh.at[di_v.at[j]], add=True)
            return carry

        lax.fori_loop(0, K16, body, 0)
        plsc.subcore_barrier()
        pltpu.sync_copy(dg_sh.at[pl.ds(sid * rpt, rpt)],
                        out_hbm.at[cid, pl.ds(sid * rpt, rpt)])

    return k(didx2, zinit)


def _k7_body(ml_ref, ids_ref, o_ref):
    i = pl.program_id(0)

    @pl.when(i == 0)
    def _():
        o_ref[...] = jnp.full((B, 256), -jnp.inf, jnp.float32)

    ninf = jnp.float32(-jnp.inf)
    iota24 = lax.broadcasted_iota(jnp.int32, (1, 24), 1)
    for g in range(RB // 8):
        row0 = 8 * g
        rr, cc = row0 // 128, row0 % 128
        id0 = ids_ref[0, rr, cc]
        w0 = pl.multiple_of(jnp.minimum((id0 // 8) * 8, B - 24), 8)
        rel = ids_ref[0, rr, pl.ds(cc, 8)] - w0
        m = rel[:, None] == iota24
        rows_g = ml_ref[pl.ds(row0, 8), :]
        cs = [jnp.max(jnp.where(m[:, s:s + 1], rows_g, ninf), axis=0,
                      keepdims=True) for s in range(24)]
        cwin = jnp.concatenate(cs, axis=0)
        o_ref[pl.ds(w0, 24), :] = jnp.maximum(o_ref[pl.ds(w0, 24), :], cwin)


def _k7(ml, bat2):
    return pl.pallas_call(
        _k7_body,
        grid=(N_P // RB,),
        in_specs=[
            pl.BlockSpec((RB, 256), lambda i: (i, 0)),
            pl.BlockSpec((1, RB // 128, 128), lambda i: (i, 0, 0)),
        ],
        out_specs=pl.BlockSpec((B, 256), lambda i: (0, 0)),
        out_shape=jax.ShapeDtypeStruct((B, 256), jnp.float32),
    )(ml, bat2)


def kernel(x, edge_index, batch, W1, b1, Wmu, bmu, Wls, bls):
    src = edge_index[0]
    dst = edge_index[1]
    x_p = jnp.pad(x, ((0, N_P - N), (0, 0)))
    batch_p = jnp.pad(batch, (0, N_P - N), constant_values=B)

    padE = jnp.full((E16 - E,), N, jnp.int32)
    srcB = jnp.concatenate([src, padE]).reshape(NS, K16, ECH)
    dst_p = jnp.concatenate([dst, padE])
    dstB = dst_p.reshape(NS, K16, ECH)
    dloc0 = jnp.where(dst_p < HALF, dst_p, HALF)
    dloc1 = jnp.where(dst_p >= HALF, dst_p - HALF, HALF)
    didxA = jnp.stack([dloc0.reshape(NS, K16, ECH), dloc1.reshape(NS, K16, ECH)])
    srcC = jnp.concatenate([src, padE]).reshape(2, NS, K16 // 2, ECH)
    dstC = dst_p.reshape(2, NS, K16 // 2, ECH)
    sidxB = jnp.stack([srcB, srcB + N_P])
    didxB = jnp.stack([dstB, dstB])

    degs = _deg_sc(didxA, jnp.zeros((2, AR, 16), jnp.float32))
    xs = _k2(x_p, degs)
    initC = jnp.stack([xs, jnp.zeros((N_P, 128), jnp.float32)])
    y1_st = _agg_sc(xs, initC, srcC, dstC, K16 // 2, N_P)
    hs_st = _k4(y1_st, degs, W1, b1)
    y2_st = _agg_sc(hs_st.reshape(2 * N_P, 128), None, sidxB, didxB,
                    K16, N_P, init_in_table=True)
    ml = _k6(y2_st, degs, Wmu, bmu, Wls, bls)
    ml_g = _k7(ml, batch_p.reshape(N_P // RB, RB // 128, 128))
    return (ml_g[:, :128], ml_g[:, 128:])


# speedup vs baseline: 7.3642x; 1.0006x over previous
import functools

import jax
import jax.numpy as jnp
from jax import lax
from jax.experimental import pallas as pl
from jax.experimental.pallas import tpu as pltpu
from jax.experimental.pallas import tpu_sc as plsc

N = 10000
E = 320000
B = 256
D_IN = 128
D_HID = 128
MAX_LOGSTD = 10.0

N_P = 10240
RB = 512


def _k2_body(x_ref, dg_ref, o_ref):
    deg = 1.0 + dg_ref[0, :, 0]
    dis = lax.rsqrt(deg)[:, None]
    o_ref[...] = x_ref[...] * dis


def _k2(x_p, degs):
    return pl.pallas_call(
        _k2_body,
        grid=(N_P // RB,),
        in_specs=[
            pl.BlockSpec((RB, 128), lambda i: (i, 0)),
            pl.BlockSpec((1, RB, 16), lambda i: (i // 10, i % 10, 0)),
        ],
        out_specs=pl.BlockSpec((RB, 128), lambda i: (i, 0)),
        out_shape=jax.ShapeDtypeStruct((N_P, 128), jnp.float32),
    )(x_p, degs)


def _k4_body(y_ref, dg_ref, w_ref, b_ref, o_ref):
    deg = 1.0 + dg_ref[0, :, 0]
    dis = lax.rsqrt(deg)[:, None]
    y1 = (y_ref[0] + y_ref[1]) * dis
    h = jnp.maximum(jnp.dot(y1, w_ref[...],
                            preferred_element_type=jnp.float32) + b_ref[...], 0.0)
    hs = h * dis
    o_ref[0] = hs[:, :128]
    o_ref[1] = hs[:, 128:]


def _k4(y1_st, degs, W1, b1):
    return pl.pallas_call(
        _k4_body,
        grid=(N_P // RB,),
        in_specs=[
            pl.BlockSpec((2, RB, 128), lambda i: (0, i, 0)),
            pl.BlockSpec((1, RB, 16), lambda i: (i // 10, i % 10, 0)),
            pl.BlockSpec((128, 256), lambda i: (0, 0)),
            pl.BlockSpec((1, 256), lambda i: (0, 0)),
        ],
        out_specs=pl.BlockSpec((2, RB, 128), lambda i: (0, i, 0)),
        out_shape=jax.ShapeDtypeStruct((2, N_P, 128), jnp.float32),
    )(y1_st, degs, W1, b1.reshape(1, 256))


def _k6_body(y_ref, dg_ref, wm_ref, bm_ref, wl_ref, bl_ref, o_ref):
    deg = 1.0 + dg_ref[0, :, 0]
    dis = lax.rsqrt(deg)[:, None]
    ah = jnp.concatenate([y_ref[0], y_ref[1]], axis=1) * dis
    mu = jnp.dot(ah, wm_ref[...], preferred_element_type=jnp.float32) + bm_ref[...]
    ls = jnp.minimum(
        jnp.dot(ah, wl_ref[...], preferred_element_type=jnp.float32) + bl_ref[...],
        MAX_LOGSTD)
    o_ref[...] = jnp.concatenate([mu, ls], axis=1)


def _k6(y2_st, degs, Wmu, bmu, Wls, bls):
    return pl.pallas_call(
        _k6_body,
        grid=(N_P // RB,),
        in_specs=[
            pl.BlockSpec((2, RB, 128), lambda i: (0, i, 0)),
            pl.BlockSpec((1, RB, 16), lambda i: (i // 10, i % 10, 0)),
            pl.BlockSpec((256, 128), lambda i: (0, 0)),
            pl.BlockSpec((1, 128), lambda i: (0, 0)),
            pl.BlockSpec((256, 128), lambda i: (0, 0)),
            pl.BlockSpec((1, 128), lambda i: (0, 0)),
        ],
        out_specs=pl.BlockSpec((RB, 256), lambda i: (i, 0)),
        out_shape=jax.ShapeDtypeStruct((N_P, 256), jnp.float32),
    )(y2_st, degs, Wmu, bmu.reshape(1, 128), Wls, bls.reshape(1, 128))


NC = 2
NS = 16
ECH = 128
K16 = 160
IBLK = 16
NBLK = K16 // IBLK
E16 = NS * K16 * ECH
HALF = N_P // 2
AR = HALF + 128


def _agg_sc(table, init, sidx2, didx2, nch, ir, init_in_table=False):
    rpt = ir // NS
    mesh = plsc.VectorSubcoreMesh(core_axis_name="c", subcore_axis_name="s")

    def _body(vs_hbm, in_hbm, si_hbm, di_hbm, out_hbm, si_v, di_v, buf, buf2,
              y_sh, sem, sem2):
        del buf2, sem2
        cid = lax.axis_index("c")
        sid = lax.axis_index("s")
        if in_hbm is None:
            pltpu.sync_copy(vs_hbm.at[pl.ds(cid * ir + sid * rpt, rpt)],
                            y_sh.at[pl.ds(sid * rpt, rpt)])
        else:
            pltpu.sync_copy(in_hbm.at[cid, pl.ds(sid * rpt, rpt)],
                            y_sh.at[pl.ds(sid * rpt, rpt)])
        plsc.subcore_barrier()

        def blk(bj, carry):
            pltpu.sync_copy(si_hbm.at[cid, sid, pl.ds(bj * IBLK, IBLK)], si_v)
            pltpu.sync_copy(di_hbm.at[cid, sid, pl.ds(bj * IBLK, IBLK)], di_v)
            def body(j, c2):
                pltpu.async_copy(vs_hbm.at[si_v.at[j]], buf, sem).wait()
                pltpu.sync_copy(buf, y_sh.at[di_v.at[j]], add=True)
                return c2

            lax.fori_loop(0, IBLK, body, 0)
            return carry

        lax.fori_loop(0, nch // IBLK, blk, 0)
        plsc.subcore_barrier()
        pltpu.sync_copy(y_sh.at[pl.ds(sid * rpt, rpt)],
                        out_hbm.at[cid, pl.ds(sid * rpt, rpt)])

    scratch = [
        pltpu.VMEM((IBLK, ECH), jnp.int32),
        pltpu.VMEM((IBLK, ECH), jnp.int32),
        pltpu.VMEM((ECH, 128), jnp.float32),
        pltpu.VMEM((ECH, 128), jnp.float32),
        pltpu.VMEM_SHARED((ir, 128), jnp.float32),
        pltpu.SemaphoreType.DMA,
        pltpu.SemaphoreType.DMA,
    ]
    out_t = jax.ShapeDtypeStruct((2, ir, 128), jnp.float32)
    if init_in_table:
        def k_body(vs, si, di, out, *s):
            _body(vs, None, si, di, out, *s)
        k = pl.kernel(k_body, out_type=out_t, mesh=mesh, scratch_types=scratch)
        return k(table, sidx2, didx2)
    k = pl.kernel(_body, out_type=out_t, mesh=mesh, scratch_types=scratch)
    return k(table, init, sidx2, didx2)


def _deg_sc(didx2, zinit):
    rpt = AR // NS
    mesh = plsc.VectorSubcoreMesh(core_axis_name="c", subcore_axis_name="s")

    @functools.partial(
        pl.kernel,
        out_type=jax.ShapeDtypeStruct((2, AR, 16), jnp.float32),
        mesh=mesh,
        scratch_types=[
            pltpu.VMEM((K16, ECH), jnp.int32),
            pltpu.VMEM((ECH, 16), jnp.float32),
            pltpu.VMEM_SHARED((AR, 16), jnp.float32),
        ],
    )
    def k(di_hbm, z_hbm, out_hbm, di_v, ones_v, dg_sh):
        cid = lax.axis_index("c")
        sid = lax.axis_index("s")
        pltpu.sync_copy(di_hbm.at[cid, sid], di_v)

        def fill(i, carry):
            ones_v[i] = jnp.full((16,), 1.0, jnp.float32)
            return carry

        lax.fori_loop(0, ECH, fill, 0)
        pltpu.sync_copy(z_hbm.at[cid, pl.ds(sid * rpt, rpt)],
                        dg_sh.at[pl.ds(sid * rpt, rpt)])
        plsc.subcore_barrier()

        def body(j, carry):
            pltpu.sync_copy(ones_v, dg_sh.at[di_v.at[j]], add=True)
            return carry

        lax.fori_loop(0, K16, body, 0)
        plsc.subcore_barrier()
        pltpu.sync_copy(dg_sh.at[pl.ds(sid * rpt, rpt)],
                        out_hbm.at[cid, pl.ds(sid * rpt, rpt)])

    return k(didx2, zinit)


def _k7_body(ml_ref, ids_ref, o_ref):
    i = pl.program_id(0)

    @pl.when(i == 0)
    def _():
        o_ref[...] = jnp.full((B, 256), -jnp.inf, jnp.float32)

    ninf = jnp.float32(-jnp.inf)
    iota24 = lax.broadcasted_iota(jnp.int32, (1, 24), 1)
    for g in range(RB // 8):
        row0 = 8 * g
        rr, cc = row0 // 128, row0 % 128
        id0 = ids_ref[0, rr, cc]
        w0 = pl.multiple_of(jnp.minimum((id0 // 8) * 8, B - 24), 8)
        rel = ids_ref[0, rr, pl.ds(cc, 8)] - w0
        m = rel[:, None] == iota24
        rows_g = ml_ref[pl.ds(row0, 8), :]
        cs = [jnp.max(jnp.where(m[:, s:s + 1], rows_g, ninf), axis=0,
                      keepdims=True) for s in range(24)]
        cwin = jnp.concatenate(cs, axis=0)
        o_ref[pl.ds(w0, 24), :] = jnp.maximum(o_ref[pl.ds(w0, 24), :], cwin)


def _k7(ml, bat2):
    return pl.pallas_call(
        _k7_body,
        grid=(N_P // RB,),
        in_specs=[
            pl.BlockSpec((RB, 256), lambda i: (i, 0)),
            pl.BlockSpec((1, RB // 128, 128), lambda i: (i, 0, 0)),
        ],
        out_specs=pl.BlockSpec((B, 256), lambda i: (0, 0)),
        out_shape=jax.ShapeDtypeStruct((B, 256), jnp.float32),
    )(ml, bat2)


def kernel(x, edge_index, batch, W1, b1, Wmu, bmu, Wls, bls):
    src = edge_index[0]
    dst = edge_index[1]
    x_p = jnp.pad(x, ((0, N_P - N), (0, 0)))
    batch_p = jnp.pad(batch, (0, N_P - N), constant_values=B)

    padE = jnp.full((E16 - E,), N, jnp.int32)
    srcB = jnp.concatenate([src, padE]).reshape(NS, K16, ECH)
    dst_p = jnp.concatenate([dst, padE])
    dstB = dst_p.reshape(NS, K16, ECH)
    dloc0 = jnp.where(dst_p < HALF, dst_p, HALF)
    dloc1 = jnp.where(dst_p >= HALF, dst_p - HALF, HALF)
    didxA = jnp.stack([dloc0.reshape(NS, K16, ECH), dloc1.reshape(NS, K16, ECH)])
    srcC = jnp.concatenate([src, padE]).reshape(2, NS, K16 // 2, ECH)
    dstC = dst_p.reshape(2, NS, K16 // 2, ECH)
    sidxB = jnp.stack([srcB, srcB + N_P])
    didxB = jnp.stack([dstB, dstB])

    degs = _deg_sc(didxA, jnp.zeros((2, AR, 16), jnp.float32))
    xs = _k2(x_p, degs)
    initC = jnp.stack([xs, jnp.zeros((N_P, 128), jnp.float32)])
    y1_st = _agg_sc(xs, initC, srcC, dstC, K16 // 2, N_P)
    hs_st = _k4(y1_st, degs, W1, b1)
    y2_st = _agg_sc(hs_st.reshape(2 * N_P, 128), None, sidxB, didxB,
                    K16, N_P, init_in_table=True)
    ml = _k6(y2_st, degs, Wmu, bmu, Wls, bls)
    ml_g = _k7(ml, batch_p.reshape(N_P // RB, RB // 128, 128))
    return (ml_g[:, :128], ml_g[:, 128:])
